# baseline (speedup 1.0000x reference)
import jax
import jax.numpy as jnp
from jax.experimental import pallas as pl

B, S, D = 4, 2048, 1024
ROWS_BLK = 1024
NH = S // ROWS_BLK


def _add_body(x_ref, pe_ref, out_ref):
    out_ref[...] = x_ref[...] + pe_ref[...]


def kernel(x, pe_table):
    batch, seq_len, d_model = x.shape
    pe = pe_table[:seq_len]
    x2 = x.reshape(batch * seq_len, d_model)
    out = pl.pallas_call(
        _add_body,
        grid=(NH, batch),
        in_specs=[
            pl.BlockSpec((ROWS_BLK, d_model), lambda h, b: (b * NH + h, 0)),
            pl.BlockSpec((ROWS_BLK, d_model), lambda h, b: (h, 0)),
        ],
        out_specs=pl.BlockSpec((ROWS_BLK, d_model), lambda h, b: (b * NH + h, 0)),
        out_shape=jax.ShapeDtypeStruct((batch * seq_len, d_model), x.dtype),
    )(x2, pe)
    return out.reshape(batch, seq_len, d_model)
